# baseline (speedup 1.0000x reference)
import functools

import jax
import jax.numpy as jnp
import numpy as np
from jax import lax
from jax.experimental import pallas as pl
from jax.experimental.pallas import tpu as pltpu
from jax.experimental.pallas import tpu_sc as plsc

NN = 10000
DD = 128
EE = 320000

NTILES = 32
CK = 112
CHUNKS = 90
E_PAD = NTILES * CHUNKS * CK
N_PAD = 10112
RPT = N_PAD // 16
DUMMY = NN
NB = 3

_mesh = plsc.VectorSubcoreMesh(core_axis_name="c", subcore_axis_name="s")


@functools.partial(
    pl.kernel,
    out_type=jax.ShapeDtypeStruct((2, N_PAD, DD), jnp.float32),
    mesh=_mesh,
    scratch_types=[
        pltpu.VMEM_SHARED((N_PAD, DD), jnp.float32),
        [pltpu.VMEM((1, CK), jnp.int32) for _ in range(NB)],
        [pltpu.VMEM((1, CK), jnp.int32) for _ in range(NB)],
        [pltpu.VMEM((CK, DD), jnp.float32) for _ in range(NB)],
        [pltpu.SemaphoreType.DMA for _ in range(NB)],
        [pltpu.SemaphoreType.DMA for _ in range(NB)],
        [pltpu.SemaphoreType.DMA for _ in range(NB)],
        [pltpu.SemaphoreType.DMA for _ in range(NB)],
    ],
)
def _sc_agg(h_hbm, z_hbm, srcs_hbm, dsts_hbm, out_hbm, agg_sh, sib, dib,
            rows, isems, jsems, gsems, ssems):
    c = lax.axis_index("c")
    s = lax.axis_index("s")
    wid = c * 16 + s
    nh = h_hbm.shape[0]

    @pl.when(c == 1)
    def _():
        pltpu.sync_copy(
            z_hbm.at[pl.ds(s * RPT, RPT)],
            agg_sh.at[pl.ds(s * RPT, RPT)],
        )

    if nh >= N_PAD:
        @pl.when(c == 0)
        def _():
            pltpu.sync_copy(
                h_hbm.at[pl.ds(s * RPT, RPT)],
                agg_sh.at[pl.ds(s * RPT, RPT)],
            )
    else:
        tail = NN - 15 * RPT

        @pl.when((c == 0) & (s < 15))
        def _():
            pltpu.sync_copy(
                h_hbm.at[pl.ds(s * RPT, RPT)],
                agg_sh.at[pl.ds(s * RPT, RPT)],
            )

        @pl.when((c == 0) & (s == 15))
        def _():
            pltpu.sync_copy(
                h_hbm.at[pl.ds(15 * RPT, tail)],
                agg_sh.at[pl.ds(15 * RPT, tail)],
            )

    plsc.subcore_barrier()

    def sidx_load(j, slot):
        pltpu.async_copy(srcs_hbm.at[wid, j], sib[slot], isems[slot])

    def sidx_wait(slot):
        pltpu.make_async_copy(srcs_hbm.at[wid, 0], sib[slot],
                              isems[slot]).wait()

    def didx_load(j, slot):
        pltpu.async_copy(dsts_hbm.at[wid, j], dib[slot], jsems[slot])

    def didx_wait(slot):
        pltpu.make_async_copy(dsts_hbm.at[wid, 0], dib[slot],
                              jsems[slot]).wait()

    def gather_start(slot):
        pltpu.async_copy(h_hbm.at[sib[slot].at[0]], rows[slot], gsems[slot])

    def gather_wait(slot):
        pltpu.make_async_copy(h_hbm.at[sib[slot].at[0]], rows[slot],
                              gsems[slot]).wait()

    def scatter_start(slot):
        pltpu.async_copy(rows[slot], agg_sh.at[dib[slot].at[0]], ssems[slot],
                         add=True)

    def scatter_wait(slot):
        pltpu.make_async_copy(rows[slot], agg_sh.at[dib[slot].at[0]],
                              ssems[slot]).wait()

    sidx_load(0, 0)
    sidx_load(1, 1)
    didx_load(0, 0)
    sidx_wait(0)
    gather_start(0)
    sidx_wait(1)
    gather_start(1)
    sidx_load(2, 2)
    sidx_wait(2)
    gather_start(2)

    def step(i, carry):
        for b in range(NB):
            j = NB * i + b
            bn = (b + 1) % NB
            bp = (b + 2) % NB
            gather_wait(b)
            didx_wait(b)
            scatter_start(b)

            scatter_wait(b)

            @pl.when(j + 3 < CHUNKS)
            def _():
                sidx_load(j + 3, b)
                sidx_wait(b)
                gather_start(b)

            @pl.when(j + 1 < CHUNKS)
            def _():
                didx_load(j + 1, bn)
        return carry

    lax.fori_loop(0, CHUNKS // NB, step, 0)
    plsc.subcore_barrier()
    pltpu.sync_copy(
        agg_sh.at[pl.ds(s * RPT, RPT)],
        out_hbm.at[c, pl.ds(s * RPT, RPT)],
    )


def _mm_body(p_ref, w_ref, b_ref, o_ref):
    rst = p_ref[0] + p_ref[1]
    o_ref[...] = (
        jnp.dot(rst, w_ref[...], preferred_element_type=jnp.float32) + b_ref[...]
    )


def _tc_mm(parts, w, b, rows_out, bm):
    return pl.pallas_call(
        _mm_body,
        grid=(rows_out // bm,),
        in_specs=[
            pl.BlockSpec((2, bm, DD), lambda i: (0, i, 0)),
            pl.BlockSpec((DD, DD), lambda i: (0, 0)),
            pl.BlockSpec((1, DD), lambda i: (0, 0)),
        ],
        out_specs=pl.BlockSpec((bm, DD), lambda i: (i, 0)),
        out_shape=jax.ShapeDtypeStruct((rows_out, DD), jnp.float32),
    )(parts, w, b.reshape(1, DD))


def kernel(x, edge_index, W1, b1, W2, b2, W3, b3):
    pad = E_PAD - EE
    pad_src = jnp.asarray(np.arange(pad, dtype=np.int32) % NN)
    pad_dst = jnp.asarray(DUMMY + np.arange(pad, dtype=np.int32) % (N_PAD - NN))
    src = jnp.concatenate([edge_index[0], pad_src])
    dst = jnp.concatenate([edge_index[1], pad_dst])
    srcs = src.reshape(NTILES, CHUNKS, 1, CK)
    dsts = dst.reshape(NTILES, CHUNKS, 1, CK)

    zeros = jnp.zeros((N_PAD, DD), jnp.float32)
    h = x
    for li, (w, b) in enumerate(((W1, b1), (W2, b2), (W3, b3))):
        parts = _sc_agg(h, zeros, srcs, dsts)
        if li < 2:
            h = _tc_mm(parts, w, b, N_PAD, N_PAD // 16)
        else:
            h = _tc_mm(parts, w, b, NN, 1000)
    return h


# speedup vs baseline: 1.5332x; 1.5332x over previous
import functools

import jax
import jax.numpy as jnp
import numpy as np
from jax import lax
from jax.experimental import pallas as pl
from jax.experimental.pallas import tpu as pltpu
from jax.experimental.pallas import tpu_sc as plsc

NN = 10000
DD = 128
EE = 320000

NTILES = 32
CK = 112
CHUNKS = 90
E_PAD = NTILES * CHUNKS * CK
N_PAD = 10112
RPT = N_PAD // 16
DUMMY = NN
NB = 3

_mesh = plsc.VectorSubcoreMesh(core_axis_name="c", subcore_axis_name="s")


@functools.partial(
    pl.kernel,
    out_type=jax.ShapeDtypeStruct((2, N_PAD, DD), jnp.float32),
    mesh=_mesh,
    scratch_types=[
        pltpu.VMEM_SHARED((N_PAD, DD), jnp.float32),
        [pltpu.VMEM((1, CK), jnp.int32) for _ in range(NB)],
        [pltpu.VMEM((1, CK), jnp.int32) for _ in range(NB)],
        [pltpu.VMEM((CK, DD), jnp.float32) for _ in range(NB)],
        [pltpu.SemaphoreType.DMA for _ in range(NB)],
        [pltpu.SemaphoreType.DMA for _ in range(NB)],
        [pltpu.SemaphoreType.DMA for _ in range(NB)],
        [pltpu.SemaphoreType.DMA for _ in range(NB)],
    ],
)
def _sc_agg(h_hbm, z_hbm, srcs_hbm, dsts_hbm, out_hbm, agg_sh, sib, dib,
            rows, isems, jsems, gsems, ssems):
    c = lax.axis_index("c")
    s = lax.axis_index("s")
    wid = c * 16 + s
    nh = h_hbm.shape[0]

    @pl.when(c == 1)
    def _():
        pltpu.sync_copy(
            z_hbm.at[pl.ds(s * RPT, RPT)],
            agg_sh.at[pl.ds(s * RPT, RPT)],
        )

    if nh >= N_PAD:
        @pl.when(c == 0)
        def _():
            pltpu.sync_copy(
                h_hbm.at[pl.ds(s * RPT, RPT)],
                agg_sh.at[pl.ds(s * RPT, RPT)],
            )
    else:
        tail = NN - 15 * RPT

        @pl.when((c == 0) & (s < 15))
        def _():
            pltpu.sync_copy(
                h_hbm.at[pl.ds(s * RPT, RPT)],
                agg_sh.at[pl.ds(s * RPT, RPT)],
            )

        @pl.when((c == 0) & (s == 15))
        def _():
            pltpu.sync_copy(
                h_hbm.at[pl.ds(15 * RPT, tail)],
                agg_sh.at[pl.ds(15 * RPT, tail)],
            )

    plsc.subcore_barrier()

    def sidx_load(j, slot):
        pltpu.async_copy(srcs_hbm.at[wid, j], sib[slot], isems[slot])

    def sidx_wait(slot):
        pltpu.make_async_copy(srcs_hbm.at[wid, 0], sib[slot],
                              isems[slot]).wait()

    def didx_load(j, slot):
        pltpu.async_copy(dsts_hbm.at[wid, j], dib[slot], jsems[slot])

    def didx_wait(slot):
        pltpu.make_async_copy(dsts_hbm.at[wid, 0], dib[slot],
                              jsems[slot]).wait()

    def gather_start(slot):
        pltpu.async_copy(h_hbm.at[sib[slot].at[0]], rows[slot], gsems[slot])

    def gather_wait(slot):
        pltpu.make_async_copy(h_hbm.at[sib[slot].at[0]], rows[slot],
                              gsems[slot]).wait()

    def scatter_start(slot):
        pltpu.async_copy(rows[slot], agg_sh.at[dib[slot].at[0]], ssems[slot],
                         add=True)

    def scatter_wait(slot):
        pltpu.make_async_copy(rows[slot], agg_sh.at[dib[slot].at[0]],
                              ssems[slot]).wait()

    sidx_load(0, 0)
    sidx_load(1, 1)
    didx_load(0, 0)
    sidx_wait(0)
    gather_start(0)
    sidx_wait(1)
    gather_start(1)
    sidx_load(2, 2)

    def step(i, carry):
        for b in range(NB):
            j = NB * i + b
            bn = (b + 1) % NB
            bp = (b + 2) % NB
            gather_wait(b)
            didx_wait(b)
            scatter_start(b)

            if b == 0:
                @pl.when(i > 0)
                def _():
                    scatter_wait(bp)
            else:
                scatter_wait(bp)

            @pl.when(j + 2 < CHUNKS)
            def _():
                sidx_wait(bp)
                gather_start(bp)

            @pl.when(j + 3 < CHUNKS)
            def _():
                sidx_load(j + 3, b)

            @pl.when(j + 1 < CHUNKS)
            def _():
                didx_load(j + 1, bn)
        return carry

    lax.fori_loop(0, CHUNKS // NB, step, 0)
    scatter_wait((CHUNKS - 1) % NB)
    plsc.subcore_barrier()
    pltpu.sync_copy(
        agg_sh.at[pl.ds(s * RPT, RPT)],
        out_hbm.at[c, pl.ds(s * RPT, RPT)],
    )


def _mm_body(p_ref, w_ref, b_ref, o_ref):
    rst = p_ref[0] + p_ref[1]
    o_ref[...] = (
        jnp.dot(rst, w_ref[...], preferred_element_type=jnp.float32) + b_ref[...]
    )


def _tc_mm(parts, w, b, rows_out, bm):
    return pl.pallas_call(
        _mm_body,
        grid=(rows_out // bm,),
        in_specs=[
            pl.BlockSpec((2, bm, DD), lambda i: (0, i, 0)),
            pl.BlockSpec((DD, DD), lambda i: (0, 0)),
            pl.BlockSpec((1, DD), lambda i: (0, 0)),
        ],
        out_specs=pl.BlockSpec((bm, DD), lambda i: (i, 0)),
        out_shape=jax.ShapeDtypeStruct((rows_out, DD), jnp.float32),
    )(parts, w, b.reshape(1, DD))


def kernel(x, edge_index, W1, b1, W2, b2, W3, b3):
    pad = E_PAD - EE
    pad_src = jnp.asarray(np.arange(pad, dtype=np.int32) % NN)
    pad_dst = jnp.asarray(DUMMY + np.arange(pad, dtype=np.int32) % (N_PAD - NN))
    src = jnp.concatenate([edge_index[0], pad_src])
    dst = jnp.concatenate([edge_index[1], pad_dst])
    srcs = src.reshape(NTILES, CHUNKS, 1, CK)
    dsts = dst.reshape(NTILES, CHUNKS, 1, CK)

    zeros = jnp.zeros((N_PAD, DD), jnp.float32)
    h = x
    for li, (w, b) in enumerate(((W1, b1), (W2, b2), (W3, b3))):
        parts = _sc_agg(h, zeros, srcs, dsts)
        if li < 2:
            h = _tc_mm(parts, w, b, N_PAD, N_PAD // 8)
        else:
            h = _tc_mm(parts, w, b, NN, 1000)
    return h
